# baseline (speedup 1.0000x reference)
import math

import jax
import jax.numpy as jnp
from jax import lax
from jax.experimental import pallas as pl
from jax.experimental.pallas import tpu as pltpu
from jax.experimental.pallas import tpu_sc as plsc

VOCAB = 1000000
EMBED_SIZE = 32
BATCH = 4096
HIST = 200
SCALE = math.sqrt(EMBED_SIZE)

NC = 2
NS = 16
NW = NC * NS
LANES = 16

BPW = BATCH // NW
HB = 8
N_HB = HIST // HB


def _body(xt_hbm, lut4_hbm, out_hbm, idx_v, idx4_v, colb_v, rows_v, tr_v,
          gsem, osem):
    wid = lax.axis_index("s") * NC + lax.axis_index("c")
    b0 = wid * BPW

    iota16 = lax.iota(jnp.int32, LANES)

    def hblock(hb, carry):
        pltpu.sync_copy(xt_hbm.at[pl.ds(hb * HB, HB), pl.ds(b0, BPW)], idx_v)
        for hh in range(HB):
            def split(k, kc):
                v = idx_v[hh, pl.ds(16 * k, LANES)]
                idx4_v[pl.ds(16 * k, LANES)] = lax.shift_right_logical(v, 2)
                colb_v[pl.ds(16 * k, LANES)] = (v & 3) * EMBED_SIZE
                return kc

            lax.fori_loop(0, 8, split, 0)
            pltpu.async_copy(lut4_hbm.at[idx4_v], rows_v, gsem).wait()

            def kblock(k, kc):
                cb = colb_v[pl.ds(16 * k, LANES)]
                ri = iota16 + 16 * k
                for e in range(EMBED_SIZE):
                    g = plsc.load_gather(rows_v, [ri, cb + e])
                    tr_v[e, pl.ds(16 * k, LANES)] = g * SCALE
                return kc

            lax.fori_loop(0, 8, kblock, 0)
            h = hb * HB + hh
            pltpu.sync_copy(tr_v, out_hbm.at[h, :, pl.ds(b0, BPW)])
        return carry

    lax.fori_loop(0, N_HB, hblock, 0)


@jax.jit
def _lookup(x_t, lut4):
    mesh = plsc.VectorSubcoreMesh(core_axis_name="c", subcore_axis_name="s")
    return pl.kernel(
        _body,
        out_type=jax.ShapeDtypeStruct((HIST, EMBED_SIZE, BATCH), jnp.float32),
        mesh=mesh,
        scratch_types=[
            pltpu.VMEM((HB, BPW), jnp.int32),
            pltpu.VMEM((BPW,), jnp.int32),
            pltpu.VMEM((BPW,), jnp.int32),
            pltpu.VMEM((BPW, 128), jnp.float32),
            pltpu.VMEM((EMBED_SIZE, BPW), jnp.float32),
            pltpu.SemaphoreType.DMA,
            pltpu.SemaphoreType.DMA,
        ],
        compiler_params=pltpu.CompilerParams(
            use_tc_tiling_on_sc=True, needs_layout_passes=False),
    )(x_t, lut4)


def kernel(x, lut):
    x_t = jnp.swapaxes(x, 0, 1)
    lut4 = lut.reshape(VOCAB // 4, 128)
    out = _lookup(x_t, lut4)
    return jnp.transpose(out, (2, 0, 1))


# speedup vs baseline: 1.2751x; 1.2751x over previous
import math

import jax
import jax.numpy as jnp
from jax import lax
from jax.experimental import pallas as pl
from jax.experimental.pallas import tpu as pltpu
from jax.experimental.pallas import tpu_sc as plsc

VOCAB = 1000000
EMBED_SIZE = 32
BATCH = 4096
HIST = 200
SCALE = math.sqrt(EMBED_SIZE)

NC = 2
NS = 16
NW = NC * NS
LANES = 16

BPW = BATCH // NW


def _body(xt_hbm, lut4_hbm, out_hbm, idx_all, idx4_all, rows, tr,
          gsems, osems):
    wid = lax.axis_index("s") * NC + lax.axis_index("c")
    b0 = wid * BPW

    iota16 = lax.iota(jnp.int32, LANES)

    pltpu.sync_copy(xt_hbm.at[:, pl.ds(b0, BPW)], idx_all)

    def split(i, c):
        h = i // 8
        k = i % 8
        v = idx_all[h, pl.ds(16 * k, LANES)]
        idx4_all[h, pl.ds(16 * k, LANES)] = lax.shift_right_logical(v, 2)
        return c

    lax.fori_loop(0, HIST * 8, split, 0)

    def gather_h(h, b):
        pltpu.async_copy(lut4_hbm.at[idx4_all.at[h]], rows[b], gsems[b])

    def gwait(h, b):
        pltpu.make_async_copy(lut4_hbm.at[idx4_all.at[h]], rows[b],
                              gsems[b]).wait()

    def out_copy(h, b):
        return pltpu.make_async_copy(tr[b], out_hbm.at[h, :, pl.ds(b0, BPW)],
                                     osems[b])

    for b in range(2):
        gather_h(jnp.int32(b), b)

    def step(p, carry):
        for b in range(2):
            h = 2 * p + b
            gwait(h, b)

            @pl.when(h >= 2)
            def _():
                out_copy(h - 2, b).wait()

            def kblock(k, kc):
                v = idx_all[h, pl.ds(16 * k, LANES)]
                cb = (v & 3) * EMBED_SIZE
                ri = iota16 + 16 * k
                for e in range(EMBED_SIZE):
                    g = plsc.load_gather(rows[b], [ri, cb + e])
                    tr[b][e, pl.ds(16 * k, LANES)] = g * SCALE
                return kc

            lax.fori_loop(0, 8, kblock, 0)
            out_copy(h, b).start()

            @pl.when(h + 2 < HIST)
            def _():
                gather_h(h + 2, b)
        return carry

    lax.fori_loop(0, HIST // 2, step, 0)
    for b in range(2):
        out_copy(HIST - 2 + b, b).wait()


@jax.jit
def _lookup(x_t, lut4):
    mesh = plsc.VectorSubcoreMesh(core_axis_name="c", subcore_axis_name="s")
    return pl.kernel(
        _body,
        out_type=jax.ShapeDtypeStruct((HIST, EMBED_SIZE, BATCH), jnp.float32),
        mesh=mesh,
        scratch_types=[
            pltpu.VMEM((HIST, BPW), jnp.int32),
            pltpu.VMEM((HIST, BPW), jnp.int32),
            [pltpu.VMEM((BPW, 128), jnp.float32) for _ in range(2)],
            [pltpu.VMEM((EMBED_SIZE, BPW), jnp.float32) for _ in range(2)],
            [pltpu.SemaphoreType.DMA for _ in range(2)],
            [pltpu.SemaphoreType.DMA for _ in range(2)],
        ],
        compiler_params=pltpu.CompilerParams(
            use_tc_tiling_on_sc=True, needs_layout_passes=False),
    )(x_t, lut4)


def kernel(x, lut):
    x_t = jnp.swapaxes(x, 0, 1)
    lut4 = lut.reshape(VOCAB // 4, 128)
    out = _lookup(x_t, lut4)
    return jnp.transpose(out, (2, 0, 1))


# speedup vs baseline: 1.8791x; 1.4737x over previous
import math

import jax
import jax.numpy as jnp
from jax import lax
from jax.experimental import pallas as pl
from jax.experimental.pallas import tpu as pltpu
from jax.experimental.pallas import tpu_sc as plsc

VOCAB = 1000000
EMBED_SIZE = 32
BATCH = 4096
HIST = 200
SCALE = math.sqrt(EMBED_SIZE)

NC = 2
NS = 16
NW = NC * NS
LANES = 16

BPW = BATCH // NW


def _body(xt_hbm, lut4_hbm, out_hbm, idx_all, idx4_all, colb_all, rows, tr,
          gsems, osems):
    wid = lax.axis_index("s") * NC + lax.axis_index("c")
    b0 = wid * BPW

    iota16 = lax.iota(jnp.int32, LANES)

    pltpu.sync_copy(xt_hbm.at[:, pl.ds(b0, BPW)], idx_all)

    def split(i, c):
        h = i // 8
        k = i % 8
        v = idx_all[h, pl.ds(16 * k, LANES)]
        idx4_all[h, pl.ds(16 * k, LANES)] = lax.shift_right_logical(v, 2)
        colb_all[h, pl.ds(16 * k, LANES)] = (v & 3) * EMBED_SIZE
        return c

    lax.fori_loop(0, HIST * 8, split, 0)

    def gather_h(h, b):
        pltpu.async_copy(lut4_hbm.at[idx4_all.at[h]], rows[b], gsems[b])

    def gwait(h, b):
        pltpu.make_async_copy(lut4_hbm.at[idx4_all.at[h]], rows[b],
                              gsems[b]).wait()

    def out_copy(h, b):
        return pltpu.make_async_copy(tr[b], out_hbm.at[h, :, pl.ds(b0, BPW)],
                                     osems[b])

    for b in range(2):
        gather_h(jnp.int32(b), b)

    def step(p, carry):
        for b in range(2):
            h = 2 * p + b
            gwait(h, b)

            @pl.when(h >= 2)
            def _():
                out_copy(h - 2, b).wait()

            def kblock(k, kc):
                cb = colb_all[h, pl.ds(16 * k, LANES)]
                ri = iota16 + 16 * k
                for e0 in range(EMBED_SIZE):
                    ev = (iota16 + e0) & (EMBED_SIZE - 1)
                    g = plsc.load_gather(rows[b], [ri, cb + ev])
                    plsc.store_scatter(tr[b], [ev, ri], g * SCALE)
                return kc

            lax.fori_loop(0, 8, kblock, 0)
            out_copy(h, b).start()

            @pl.when(h + 2 < HIST)
            def _():
                gather_h(h + 2, b)
        return carry

    lax.fori_loop(0, HIST // 2, step, 0)
    for b in range(2):
        out_copy(HIST - 2 + b, b).wait()


@jax.jit
def _lookup(x_t, lut4):
    mesh = plsc.VectorSubcoreMesh(core_axis_name="c", subcore_axis_name="s")
    return pl.kernel(
        _body,
        out_type=jax.ShapeDtypeStruct((HIST, EMBED_SIZE, BATCH), jnp.float32),
        mesh=mesh,
        scratch_types=[
            pltpu.VMEM((HIST, BPW), jnp.int32),
            pltpu.VMEM((HIST, BPW), jnp.int32),
            pltpu.VMEM((HIST, BPW), jnp.int32),
            [pltpu.VMEM((BPW, 128), jnp.float32) for _ in range(2)],
            [pltpu.VMEM((EMBED_SIZE, BPW), jnp.float32) for _ in range(2)],
            [pltpu.SemaphoreType.DMA for _ in range(2)],
            [pltpu.SemaphoreType.DMA for _ in range(2)],
        ],
        compiler_params=pltpu.CompilerParams(
            use_tc_tiling_on_sc=True, needs_layout_passes=False),
    )(x_t, lut4)


def kernel(x, lut):
    x_t = jnp.swapaxes(x, 0, 1)
    lut4 = lut.reshape(VOCAB // 4, 128)
    out = _lookup(x_t, lut4)
    return jnp.transpose(out, (2, 0, 1))


# speedup vs baseline: 1.8900x; 1.0058x over previous
import math

import jax
import jax.numpy as jnp
from jax import lax
from jax.experimental import pallas as pl
from jax.experimental.pallas import tpu as pltpu
from jax.experimental.pallas import tpu_sc as plsc

VOCAB = 1000000
EMBED_SIZE = 32
BATCH = 4096
HIST = 200
SCALE = math.sqrt(EMBED_SIZE)

NC = 2
NS = 16
NW = NC * NS
LANES = 16

BPW = BATCH // NW
NBUF = 3
NTR = 2


def _body(xt_hbm, lutp_hbm, out_hbm, idx_all, rows, tr, gsems, osems):
    wid = lax.axis_index("s") * NC + lax.axis_index("c")
    b0 = wid * BPW

    iota16 = lax.iota(jnp.int32, LANES)

    pltpu.sync_copy(xt_hbm.at[:, pl.ds(b0, BPW)], idx_all)

    def gather_h(h, b):
        pltpu.async_copy(lutp_hbm.at[idx_all.at[h]], rows[b], gsems[b])

    def gwait(h, b):
        pltpu.make_async_copy(lutp_hbm.at[idx_all.at[h]], rows[b],
                              gsems[b]).wait()

    def out_copy(h, t):
        return pltpu.make_async_copy(tr[t], out_hbm.at[h, :, pl.ds(b0, BPW)],
                                     osems[t])

    for b in range(NBUF):
        gather_h(jnp.int32(b), b)

    def step(p, carry):
        for q in range(NBUF * NTR):
            h = NBUF * NTR * p + q
            b = q % NBUF
            t = q % NTR
            gwait(h, b)

            @pl.when(h >= NTR)
            def _():
                out_copy(h - NTR, t).wait()

            def kblock(k, kc):
                ri = iota16 + 16 * k
                for e0 in range(EMBED_SIZE):
                    ev = (iota16 + e0) & (EMBED_SIZE - 1)
                    g = plsc.load_gather(rows[b], [ri, ev])
                    plsc.store_scatter(tr[t], [ev, ri], g * SCALE)
                return kc

            lax.fori_loop(0, 8, kblock, 0)
            out_copy(h, t).start()

            @pl.when(h + NBUF < HIST)
            def _():
                gather_h(h + NBUF, b)
        return carry

    lax.fori_loop(0, HIST // (NBUF * NTR), step, 0)

    for q in range(HIST % (NBUF * NTR)):
        h = HIST - (HIST % (NBUF * NTR)) + q
        b = q % NBUF
        t = q % NTR
        gwait(h, b)
        out_copy(h - NTR, t).wait()

        def kblock_t(k, kc):
            ri = iota16 + 16 * k
            for e0 in range(EMBED_SIZE):
                ev = (iota16 + e0) & (EMBED_SIZE - 1)
                g = plsc.load_gather(rows[b], [ri, ev])
                plsc.store_scatter(tr[t], [ev, ri], g * SCALE)
            return kc

        lax.fori_loop(0, 8, kblock_t, 0)
        out_copy(h, t).start()

    for q in range(NTR):
        out_copy(HIST - NTR + q, (HIST - NTR + q) % NTR).wait()


@jax.jit
def _lookup(x_t, lutp):
    mesh = plsc.VectorSubcoreMesh(core_axis_name="c", subcore_axis_name="s")
    return pl.kernel(
        _body,
        out_type=jax.ShapeDtypeStruct((HIST, EMBED_SIZE, BATCH), jnp.float32),
        mesh=mesh,
        scratch_types=[
            pltpu.VMEM((HIST, BPW), jnp.int32),
            [pltpu.VMEM((BPW, 128), jnp.float32) for _ in range(NBUF)],
            [pltpu.VMEM((EMBED_SIZE, BPW), jnp.float32) for _ in range(NTR)],
            [pltpu.SemaphoreType.DMA for _ in range(NBUF)],
            [pltpu.SemaphoreType.DMA for _ in range(NTR)],
        ],
        compiler_params=pltpu.CompilerParams(
            use_tc_tiling_on_sc=True, needs_layout_passes=False),
    )(x_t, lutp)


def kernel(x, lut):
    x_t = jnp.swapaxes(x, 0, 1)
    lutp = jnp.pad(lut, ((0, 0), (0, 128 - EMBED_SIZE)))
    out = _lookup(x_t, lutp)
    return jnp.transpose(out, (2, 0, 1))


# speedup vs baseline: 1.9879x; 1.0518x over previous
import math

import jax
import jax.numpy as jnp
from jax import lax
from jax.experimental import pallas as pl
from jax.experimental.pallas import tpu as pltpu
from jax.experimental.pallas import tpu_sc as plsc

VOCAB = 1000000
EMBED_SIZE = 32
BATCH = 4096
HIST = 200
SCALE = math.sqrt(EMBED_SIZE)

NC = 2
NS = 16
NW = NC * NS
LANES = 16

BPW = BATCH // NW
NBUF = 5
NTR = 2


def _body(xt_hbm, lutp_hbm, out_hbm, idx_all, rows, tr, gsems, osems):
    wid = lax.axis_index("s") * NC + lax.axis_index("c")
    b0 = wid * BPW

    iota16 = lax.iota(jnp.int32, LANES)

    pltpu.sync_copy(xt_hbm.at[:, pl.ds(b0, BPW)], idx_all)

    def gather_h(h, b):
        pltpu.async_copy(lutp_hbm.at[idx_all.at[h]], rows[b], gsems[b])

    def gwait(h, b):
        pltpu.make_async_copy(lutp_hbm.at[idx_all.at[h]], rows[b],
                              gsems[b]).wait()

    def out_copy(h, t):
        return pltpu.make_async_copy(tr[t], out_hbm.at[h, :, pl.ds(b0, BPW)],
                                     osems[t])

    for b in range(NBUF):
        gather_h(jnp.int32(b), b)

    def step(p, carry):
        for q in range(NBUF * NTR):
            h = NBUF * NTR * p + q
            b = q % NBUF
            t = q % NTR
            gwait(h, b)

            @pl.when(h >= NTR)
            def _():
                out_copy(h - NTR, t).wait()

            def kblock(k, kc):
                ri = iota16 + 16 * k
                for e0 in range(EMBED_SIZE):
                    ev = (iota16 + e0) & (EMBED_SIZE - 1)
                    g = plsc.load_gather(rows[b], [ri, ev])
                    plsc.store_scatter(tr[t], [ev, ri], g * SCALE)
                return kc

            lax.fori_loop(0, 8, kblock, 0, unroll=2)
            out_copy(h, t).start()

            @pl.when(h + NBUF < HIST)
            def _():
                gather_h(h + NBUF, b)
        return carry

    lax.fori_loop(0, HIST // (NBUF * NTR), step, 0)

    for q in range(HIST % (NBUF * NTR)):
        h = HIST - (HIST % (NBUF * NTR)) + q
        b = q % NBUF
        t = q % NTR
        gwait(h, b)
        out_copy(h - NTR, t).wait()

        def kblock_t(k, kc):
            ri = iota16 + 16 * k
            for e0 in range(EMBED_SIZE):
                ev = (iota16 + e0) & (EMBED_SIZE - 1)
                g = plsc.load_gather(rows[b], [ri, ev])
                plsc.store_scatter(tr[t], [ev, ri], g * SCALE)
            return kc

        lax.fori_loop(0, 8, kblock_t, 0)
        out_copy(h, t).start()

    for q in range(NTR):
        out_copy(HIST - NTR + q, (HIST - NTR + q) % NTR).wait()


@jax.jit
def _lookup(x_t, lutp):
    mesh = plsc.VectorSubcoreMesh(core_axis_name="c", subcore_axis_name="s")
    return pl.kernel(
        _body,
        out_type=jax.ShapeDtypeStruct((HIST, EMBED_SIZE, BATCH), jnp.float32),
        mesh=mesh,
        scratch_types=[
            pltpu.VMEM((HIST, BPW), jnp.int32),
            [pltpu.VMEM((BPW, 128), jnp.float32) for _ in range(NBUF)],
            [pltpu.VMEM((EMBED_SIZE, BPW), jnp.float32) for _ in range(NTR)],
            [pltpu.SemaphoreType.DMA for _ in range(NBUF)],
            [pltpu.SemaphoreType.DMA for _ in range(NTR)],
        ],
        compiler_params=pltpu.CompilerParams(
            use_tc_tiling_on_sc=True, needs_layout_passes=False),
    )(x_t, lutp)


def kernel(x, lut):
    x_t = jnp.swapaxes(x, 0, 1)
    lutp = jnp.pad(lut, ((0, 0), (0, 128 - EMBED_SIZE)))
    out = _lookup(x_t, lutp)
    return jnp.transpose(out, (2, 0, 1))
